# baseline (speedup 1.0000x reference)
import functools

import jax
import jax.numpy as jnp
from jax import lax
from jax.experimental import pallas as pl
from jax.experimental.pallas import tpu as pltpu
from jax.experimental.pallas import tpu_sc as plsc

_VOCAB = 30522
_HIDDEN = 768
_MAX_POS = 512
_BATCH = 10
_B = _BATCH * _MAX_POS
_NC, _NS = 2, 16
_NW = _NC * _NS
_B_PER_W = _B // _NW


def _sc_gather(word_table, flat_idx):
    mesh = plsc.VectorSubcoreMesh(core_axis_name="c", subcore_axis_name="s")

    @functools.partial(
        pl.kernel,
        mesh=mesh,
        out_type=jax.ShapeDtypeStruct((_B, _HIDDEN), jnp.float32),
        scratch_types=[
            pltpu.VMEM((_B_PER_W,), jnp.int32),
            pltpu.VMEM((_B_PER_W, _HIDDEN), jnp.float32),
            pltpu.SemaphoreType.DMA,
        ],
    )
    def gather_kernel(table_hbm, idx_hbm, out_hbm, idx_v, rows_v, sem):
        wid = lax.axis_index("s") * _NC + lax.axis_index("c")
        base = wid * _B_PER_W
        pltpu.sync_copy(idx_hbm.at[pl.ds(base, _B_PER_W)], idx_v)
        pltpu.async_copy(table_hbm.at[idx_v], rows_v, sem).wait()
        pltpu.sync_copy(rows_v, out_hbm.at[pl.ds(base, _B_PER_W)])

    return gather_kernel(word_table, flat_idx)


_TOK_BLK = 64
_N_BLKS = _B // _TOK_BLK
_POS_BLKS = _MAX_POS // _TOK_BLK


def _tc_body(word_ref, pos_ref, seg_ref, type_ref, gam_ref, bet_ref, out_ref):
    i = pl.program_id(0)
    pos_blk = pos_ref[pl.ds((i % _POS_BLKS) * _TOK_BLK, _TOK_BLK), :]
    x = word_ref[...] + pos_blk
    segc = seg_ref[:, 0:1]
    t0 = type_ref[0, :][None, :]
    t1 = type_ref[1, :][None, :]
    x = x + (t0 + segc * (t1 - t0))
    mean = jnp.mean(x, axis=1, keepdims=True)
    xc = x - mean
    var = jnp.mean(xc * xc, axis=1, keepdims=True)
    y = xc * lax.rsqrt(var + 1e-5)
    out_ref[...] = y * gam_ref[0, :][None, :] + bet_ref[0, :][None, :]


def _tc_finish(gathered, seg3, pos_table, type_table, gamma2, beta2):
    return pl.pallas_call(
        _tc_body,
        grid=(_N_BLKS,),
        in_specs=[
            pl.BlockSpec((_TOK_BLK, _HIDDEN), lambda i: (i, 0)),
            pl.BlockSpec((_MAX_POS, _HIDDEN), lambda i: (0, 0)),
            pl.BlockSpec((_TOK_BLK, 128), lambda i: (i, 0)),
            pl.BlockSpec((2, _HIDDEN), lambda i: (0, 0)),
            pl.BlockSpec((1, _HIDDEN), lambda i: (0, 0)),
            pl.BlockSpec((1, _HIDDEN), lambda i: (0, 0)),
        ],
        out_specs=pl.BlockSpec((_TOK_BLK, _HIDDEN), lambda i: (i, 0)),
        out_shape=jax.ShapeDtypeStruct((_B, _HIDDEN), jnp.float32),
    )(gathered, pos_table, seg3, type_table, gamma2, beta2)


def kernel(batch_idx, batch_seg_idx, word_table, pos_table, type_table,
           ln_gamma, ln_beta):
    flat_idx = batch_idx.reshape(-1).astype(jnp.int32)
    gathered = _sc_gather(word_table, flat_idx)
    segb = jnp.broadcast_to(
        batch_seg_idx.reshape(_B, 1).astype(jnp.float32), (_B, 128))
    out = _tc_finish(gathered, segb, pos_table, type_table,
                     ln_gamma.reshape(1, _HIDDEN), ln_beta.reshape(1, _HIDDEN))
    return out.reshape(_BATCH, _MAX_POS, _HIDDEN)


# speedup vs baseline: 1.5704x; 1.5704x over previous
import functools

import jax
import jax.numpy as jnp
from jax import lax
from jax.experimental import pallas as pl
from jax.experimental.pallas import tpu as pltpu
from jax.experimental.pallas import tpu_sc as plsc

_VOCAB = 30522
_HIDDEN = 768
_MAX_POS = 512
_BATCH = 10
_B = _BATCH * _MAX_POS
_NC, _NS = 2, 16
_NW = _NC * _NS
_B_PER_W = _B // _NW


def _sc_gather(word_table, flat_idx):
    mesh = plsc.VectorSubcoreMesh(core_axis_name="c", subcore_axis_name="s")

    @functools.partial(
        pl.kernel,
        mesh=mesh,
        out_type=jax.ShapeDtypeStruct((_B, _HIDDEN), jnp.float32),
        scratch_types=[
            pltpu.VMEM((_B_PER_W,), jnp.int32),
            pltpu.VMEM((_B_PER_W, _HIDDEN), jnp.float32),
            pltpu.SemaphoreType.DMA,
        ],
    )
    def gather_kernel(table_hbm, idx_hbm, out_hbm, idx_v, rows_v, sem):
        wid = lax.axis_index("s") * _NC + lax.axis_index("c")
        base = wid * _B_PER_W
        pltpu.sync_copy(idx_hbm.at[pl.ds(base, _B_PER_W)], idx_v)
        pltpu.async_copy(table_hbm.at[idx_v], rows_v, sem).wait()
        pltpu.sync_copy(rows_v, out_hbm.at[pl.ds(base, _B_PER_W)])

    return gather_kernel(word_table, flat_idx)


_TOK_BLK = 256
_N_BLKS = _B // _TOK_BLK
_POS_BLKS = _MAX_POS // _TOK_BLK


def _tc_body(word_ref, pos_ref, seg_ref, type_ref, gam_ref, bet_ref, out_ref):
    i = pl.program_id(0)
    pos_blk = pos_ref[pl.ds((i % _POS_BLKS) * _TOK_BLK, _TOK_BLK), :]
    x = word_ref[...] + pos_blk
    segc = seg_ref[:, 0:1]
    t0 = type_ref[0, :][None, :]
    t1 = type_ref[1, :][None, :]
    x = x + (t0 + segc * (t1 - t0))
    mean = jnp.mean(x, axis=1, keepdims=True)
    xc = x - mean
    var = jnp.mean(xc * xc, axis=1, keepdims=True)
    y = xc * lax.rsqrt(var + 1e-5)
    out_ref[...] = y * gam_ref[0, :][None, :] + bet_ref[0, :][None, :]


def _tc_finish(gathered, seg3, pos_table, type_table, gamma2, beta2):
    return pl.pallas_call(
        _tc_body,
        grid=(_N_BLKS,),
        in_specs=[
            pl.BlockSpec((_TOK_BLK, _HIDDEN), lambda i: (i, 0)),
            pl.BlockSpec((_MAX_POS, _HIDDEN), lambda i: (0, 0)),
            pl.BlockSpec((_TOK_BLK, 128), lambda i: (i, 0)),
            pl.BlockSpec((2, _HIDDEN), lambda i: (0, 0)),
            pl.BlockSpec((1, _HIDDEN), lambda i: (0, 0)),
            pl.BlockSpec((1, _HIDDEN), lambda i: (0, 0)),
        ],
        out_specs=pl.BlockSpec((_TOK_BLK, _HIDDEN), lambda i: (i, 0)),
        out_shape=jax.ShapeDtypeStruct((_B, _HIDDEN), jnp.float32),
    )(gathered, pos_table, seg3, type_table, gamma2, beta2)


def kernel(batch_idx, batch_seg_idx, word_table, pos_table, type_table,
           ln_gamma, ln_beta):
    flat_idx = batch_idx.reshape(-1).astype(jnp.int32)
    gathered = _sc_gather(word_table, flat_idx)
    segb = jnp.broadcast_to(
        batch_seg_idx.reshape(_B, 1).astype(jnp.float32), (_B, 128))
    out = _tc_finish(gathered, segb, pos_table, type_table,
                     ln_gamma.reshape(1, _HIDDEN), ln_beta.reshape(1, _HIDDEN))
    return out.reshape(_BATCH, _MAX_POS, _HIDDEN)


# speedup vs baseline: 1.7548x; 1.1174x over previous
import functools

import jax
import jax.numpy as jnp
from jax import lax
from jax.experimental import pallas as pl
from jax.experimental.pallas import tpu as pltpu
from jax.experimental.pallas import tpu_sc as plsc

_VOCAB = 30522
_HIDDEN = 768
_MAX_POS = 512
_BATCH = 10
_B = _BATCH * _MAX_POS
_NC, _NS = 2, 16
_NW = _NC * _NS
_B_PER_W = _B // _NW


def _sc_gather(word_table, flat_idx):
    mesh = plsc.VectorSubcoreMesh(core_axis_name="c", subcore_axis_name="s")

    @functools.partial(
        pl.kernel,
        mesh=mesh,
        out_type=jax.ShapeDtypeStruct((_B, _HIDDEN), jnp.float32),
        scratch_types=[
            pltpu.VMEM((_B_PER_W,), jnp.int32),
            pltpu.VMEM((_B_PER_W, _HIDDEN), jnp.float32),
            pltpu.SemaphoreType.DMA,
        ],
    )
    def gather_kernel(table_hbm, idx_hbm, out_hbm, idx_v, rows_v, sem):
        wid = lax.axis_index("s") * _NC + lax.axis_index("c")
        base = wid * _B_PER_W
        pltpu.sync_copy(idx_hbm.at[pl.ds(base, _B_PER_W)], idx_v)
        pltpu.async_copy(table_hbm.at[idx_v], rows_v, sem).wait()
        pltpu.sync_copy(rows_v, out_hbm.at[pl.ds(base, _B_PER_W)])

    return gather_kernel(word_table, flat_idx)


_TOK_BLK = 512
_N_BLKS = _B // _TOK_BLK
_POS_BLKS = _MAX_POS // _TOK_BLK


def _tc_body(word_ref, pos_ref, seg_ref, type_ref, gam_ref, bet_ref, out_ref):
    i = pl.program_id(0)
    pos_blk = pos_ref[pl.ds((i % _POS_BLKS) * _TOK_BLK, _TOK_BLK), :]
    x = word_ref[...] + pos_blk
    segc = seg_ref[:, 0:1]
    t0 = type_ref[0, :][None, :]
    t1 = type_ref[1, :][None, :]
    x = x + (t0 + segc * (t1 - t0))
    mean = jnp.mean(x, axis=1, keepdims=True)
    xc = x - mean
    var = jnp.mean(xc * xc, axis=1, keepdims=True)
    y = xc * lax.rsqrt(var + 1e-5)
    out_ref[...] = y * gam_ref[0, :][None, :] + bet_ref[0, :][None, :]


def _tc_finish(gathered, seg3, pos_table, type_table, gamma2, beta2):
    return pl.pallas_call(
        _tc_body,
        grid=(_N_BLKS,),
        in_specs=[
            pl.BlockSpec((_TOK_BLK, _HIDDEN), lambda i: (i, 0)),
            pl.BlockSpec((_MAX_POS, _HIDDEN), lambda i: (0, 0)),
            pl.BlockSpec((_TOK_BLK, 128), lambda i: (i, 0)),
            pl.BlockSpec((2, _HIDDEN), lambda i: (0, 0)),
            pl.BlockSpec((1, _HIDDEN), lambda i: (0, 0)),
            pl.BlockSpec((1, _HIDDEN), lambda i: (0, 0)),
        ],
        out_specs=pl.BlockSpec((_TOK_BLK, _HIDDEN), lambda i: (i, 0)),
        out_shape=jax.ShapeDtypeStruct((_B, _HIDDEN), jnp.float32),
    )(gathered, pos_table, seg3, type_table, gamma2, beta2)


def kernel(batch_idx, batch_seg_idx, word_table, pos_table, type_table,
           ln_gamma, ln_beta):
    flat_idx = batch_idx.reshape(-1).astype(jnp.int32)
    gathered = _sc_gather(word_table, flat_idx)
    segb = jnp.broadcast_to(
        batch_seg_idx.reshape(_B, 1).astype(jnp.float32), (_B, 128))
    out = _tc_finish(gathered, segb, pos_table, type_table,
                     ln_gamma.reshape(1, _HIDDEN), ln_beta.reshape(1, _HIDDEN))
    return out.reshape(_BATCH, _MAX_POS, _HIDDEN)


# speedup vs baseline: 1.7768x; 1.0126x over previous
import functools

import jax
import jax.numpy as jnp
from jax import lax
from jax.experimental import pallas as pl
from jax.experimental.pallas import tpu as pltpu
from jax.experimental.pallas import tpu_sc as plsc

_VOCAB = 30522
_HIDDEN = 768
_MAX_POS = 512
_BATCH = 10
_B = _BATCH * _MAX_POS
_NC, _NS = 2, 16
_NW = _NC * _NS
_B_PER_W = _B // _NW


def _sc_gather(word_table, flat_idx):
    mesh = plsc.VectorSubcoreMesh(core_axis_name="c", subcore_axis_name="s")

    @functools.partial(
        pl.kernel,
        mesh=mesh,
        out_type=jax.ShapeDtypeStruct((_B, _HIDDEN), jnp.float32),
        scratch_types=[
            pltpu.VMEM((_B_PER_W,), jnp.int32),
            pltpu.VMEM((_B_PER_W, _HIDDEN), jnp.float32),
            pltpu.SemaphoreType.DMA,
        ],
    )
    def gather_kernel(table_hbm, idx_hbm, out_hbm, idx_v, rows_v, sem):
        wid = lax.axis_index("s") * _NC + lax.axis_index("c")
        base = wid * _B_PER_W
        pltpu.sync_copy(idx_hbm.at[pl.ds(base, _B_PER_W)], idx_v)
        pltpu.async_copy(table_hbm.at[idx_v], rows_v, sem).wait()
        pltpu.sync_copy(rows_v, out_hbm.at[pl.ds(base, _B_PER_W)])

    return gather_kernel(word_table, flat_idx)


_TOK_BLK = 1024
_N_BLKS = _B // _TOK_BLK
_POS_BLKS = _MAX_POS // _TOK_BLK


def _tc_body(word_ref, pos_ref, seg_ref, type_ref, gam_ref, bet_ref, out_ref):
    pos = pos_ref[...]
    pos_blk = jnp.concatenate([pos] * (_TOK_BLK // _MAX_POS), axis=0)
    x = word_ref[...] + pos_blk
    segc = seg_ref[:, 0:1]
    t0 = type_ref[0, :][None, :]
    t1 = type_ref[1, :][None, :]
    x = x + (t0 + segc * (t1 - t0))
    mean = jnp.mean(x, axis=1, keepdims=True)
    xc = x - mean
    var = jnp.mean(xc * xc, axis=1, keepdims=True)
    y = xc * lax.rsqrt(var + 1e-5)
    out_ref[...] = y * gam_ref[0, :][None, :] + bet_ref[0, :][None, :]


def _tc_finish(gathered, seg3, pos_table, type_table, gamma2, beta2):
    return pl.pallas_call(
        _tc_body,
        grid=(_N_BLKS,),
        in_specs=[
            pl.BlockSpec((_TOK_BLK, _HIDDEN), lambda i: (i, 0)),
            pl.BlockSpec((_MAX_POS, _HIDDEN), lambda i: (0, 0)),
            pl.BlockSpec((_TOK_BLK, 128), lambda i: (i, 0)),
            pl.BlockSpec((2, _HIDDEN), lambda i: (0, 0)),
            pl.BlockSpec((1, _HIDDEN), lambda i: (0, 0)),
            pl.BlockSpec((1, _HIDDEN), lambda i: (0, 0)),
        ],
        out_specs=pl.BlockSpec((_TOK_BLK, _HIDDEN), lambda i: (i, 0)),
        out_shape=jax.ShapeDtypeStruct((_B, _HIDDEN), jnp.float32),
    )(gathered, pos_table, seg3, type_table, gamma2, beta2)


def kernel(batch_idx, batch_seg_idx, word_table, pos_table, type_table,
           ln_gamma, ln_beta):
    flat_idx = batch_idx.reshape(-1).astype(jnp.int32)
    gathered = _sc_gather(word_table, flat_idx)
    segb = jnp.broadcast_to(
        batch_seg_idx.reshape(_B, 1).astype(jnp.float32), (_B, 128))
    out = _tc_finish(gathered, segb, pos_table, type_table,
                     ln_gamma.reshape(1, _HIDDEN), ln_beta.reshape(1, _HIDDEN))
    return out.reshape(_BATCH, _MAX_POS, _HIDDEN)


# speedup vs baseline: 1.9076x; 1.0736x over previous
import functools

import jax
import jax.numpy as jnp
from jax import lax
from jax.experimental import pallas as pl
from jax.experimental.pallas import tpu as pltpu
from jax.experimental.pallas import tpu_sc as plsc

_VOCAB = 30522
_HIDDEN = 768
_MAX_POS = 512
_BATCH = 10
_B = _BATCH * _MAX_POS
_NC, _NS = 2, 16
_NW = _NC * _NS
_B_PER_W = _B // _NW


def _sc_gather(word_table, flat_idx):
    mesh = plsc.VectorSubcoreMesh(core_axis_name="c", subcore_axis_name="s")

    @functools.partial(
        pl.kernel,
        mesh=mesh,
        out_type=jax.ShapeDtypeStruct((_B, _HIDDEN), jnp.float32),
        scratch_types=[
            pltpu.VMEM((_B_PER_W,), jnp.int32),
            pltpu.VMEM((_B_PER_W, _HIDDEN), jnp.float32),
            pltpu.SemaphoreType.DMA,
        ],
    )
    def gather_kernel(table_hbm, idx_hbm, out_hbm, idx_v, rows_v, sem):
        wid = lax.axis_index("s") * _NC + lax.axis_index("c")
        base = wid * _B_PER_W
        pltpu.sync_copy(idx_hbm.at[pl.ds(base, _B_PER_W)], idx_v)
        pltpu.async_copy(table_hbm.at[idx_v], rows_v, sem).wait()
        pltpu.sync_copy(rows_v, out_hbm.at[pl.ds(base, _B_PER_W)])

    return gather_kernel(word_table, flat_idx)


_TOK_BLK = 1024
_N_BLKS = _B // _TOK_BLK
_POS_BLKS = _MAX_POS // _TOK_BLK


def _tc_body(word_ref, pos_ref, seg_ref, type_ref, gam_ref, bet_ref, out_ref):
    pos = pos_ref[...]
    pos_blk = jnp.concatenate([pos] * (_TOK_BLK // _MAX_POS), axis=0)
    x = word_ref[...] + pos_blk
    seg_row = seg_ref[0].astype(jnp.float32)
    seg_sq = jnp.broadcast_to(seg_row, (128, _TOK_BLK))
    segc = seg_sq.T[:, 0:1]
    t0 = type_ref[0, :][None, :]
    t1 = type_ref[1, :][None, :]
    x = x + (t0 + segc * (t1 - t0))
    mean = jnp.mean(x, axis=1, keepdims=True)
    xc = x - mean
    var = jnp.mean(xc * xc, axis=1, keepdims=True)
    y = xc * lax.rsqrt(var + 1e-5)
    out_ref[...] = y * gam_ref[0, :][None, :] + bet_ref[0, :][None, :]


def _tc_finish(gathered, seg3, pos_table, type_table, gamma2, beta2):
    return pl.pallas_call(
        _tc_body,
        grid=(_N_BLKS,),
        in_specs=[
            pl.BlockSpec((_TOK_BLK, _HIDDEN), lambda i: (i, 0)),
            pl.BlockSpec((_MAX_POS, _HIDDEN), lambda i: (0, 0)),
            pl.BlockSpec((1, 1, _TOK_BLK), lambda i: (i, 0, 0)),
            pl.BlockSpec((2, _HIDDEN), lambda i: (0, 0)),
            pl.BlockSpec((1, _HIDDEN), lambda i: (0, 0)),
            pl.BlockSpec((1, _HIDDEN), lambda i: (0, 0)),
        ],
        out_specs=pl.BlockSpec((_TOK_BLK, _HIDDEN), lambda i: (i, 0)),
        out_shape=jax.ShapeDtypeStruct((_B, _HIDDEN), jnp.float32),
    )(gathered, pos_table, seg3, type_table, gamma2, beta2)


def kernel(batch_idx, batch_seg_idx, word_table, pos_table, type_table,
           ln_gamma, ln_beta):
    flat_idx = batch_idx.reshape(-1).astype(jnp.int32)
    gathered = _sc_gather(word_table, flat_idx)
    segb = batch_seg_idx.reshape(_N_BLKS, 1, _TOK_BLK).astype(jnp.int32)
    out = _tc_finish(gathered, segb, pos_table, type_table,
                     ln_gamma.reshape(1, _HIDDEN), ln_beta.reshape(1, _HIDDEN))
    return out.reshape(_BATCH, _MAX_POS, _HIDDEN)
